# Initial kernel scaffold; baseline (speedup 1.0000x reference)
import jax
import jax.numpy as jnp
from jax.experimental import pallas as pl


def kernel(x, pe):
    raise NotImplementedError("write your pallas kernel here")


# speedup vs baseline: 2.5733x; 2.5733x over previous
import functools

import jax
import jax.numpy as jnp
from jax import lax
from jax.experimental import pallas as pl
from jax.experimental.pallas import tpu as pltpu
from jax.experimental.pallas import tpu_sc as plsc

D = 128
B = 4096 * 200
NW = 32
BPW = B // NW
C = 128
NCHUNK = BPW // C

_mesh = plsc.VectorSubcoreMesh(core_axis_name="c", subcore_axis_name="s")


@functools.partial(
    pl.kernel,
    mesh=_mesh,
    out_type=jax.ShapeDtypeStruct((B, D), jnp.float32),
    scratch_types=[
        pltpu.VMEM((C,), jnp.float32),
        pltpu.VMEM((C,), jnp.int32),
        pltpu.VMEM((C, D), jnp.float32),
        pltpu.SemaphoreType.DMA,
    ],
)
def _encode(x_hbm, pe_hbm, out_hbm, xbuf, idxbuf, rows, gsem):
    cid = lax.axis_index("c")
    sid = lax.axis_index("s")
    wid = sid * 2 + cid
    base0 = wid * BPW

    def body(g, carry):
        base = base0 + g * C
        pltpu.sync_copy(x_hbm.at[pl.ds(base, C)], xbuf)
        for i in range(C // 16):
            v = xbuf[pl.ds(i * 16, 16)]
            y = jnp.maximum(v, jnp.float32(0.01)) * jnp.float32(100.0)
            r = y + jnp.float32(0.5)
            t = r.astype(jnp.int32)
            tie = jnp.where(t.astype(jnp.float32) == r, t & 1, 0)
            idxbuf[pl.ds(i * 16, 16)] = t - tie - 1
        pltpu.async_copy(pe_hbm.at[idxbuf], rows, gsem).wait()
        pltpu.sync_copy(rows, out_hbm.at[pl.ds(base, C)])
        return carry

    lax.fori_loop(0, NCHUNK, body, 0)


def kernel(x, pe):
    out = _encode(x.reshape(B), pe)
    return out.reshape(x.shape[0], x.shape[1], D)
